# baseline (speedup 1.0000x reference)
import functools

import jax
import jax.numpy as jnp
import numpy as np
from jax import lax
from jax.experimental import pallas as pl
from jax.experimental.pallas import tpu as pltpu
from jax.experimental.pallas import tpu_sc as plsc

NC = 2
NS = 16
NW = NC * NS
CH = 80
NBUF = 5
GB = CH * NBUF


def _sc_mesh():
    return plsc.VectorSubcoreMesh(
        core_axis_name="c", subcore_axis_name="s", num_cores=NC, num_subcores=NS
    )


def _mm(x, w):
    return lax.dot_general(x, w, (((1,), (1,)), ((), ())),
                           preferred_element_type=jnp.float32)


def _ln(x, g, b):
    mu = jnp.mean(x, axis=-1, keepdims=True)
    var = jnp.mean((x - mu) ** 2, axis=-1, keepdims=True)
    return (x - mu) * lax.rsqrt(var + 1e-5) * g + b


def _prep_v(vfeat, w_eff, b_eff, w_kv, b_kv, w_vv, b_vv, w_qv, b_qv):
    n = vfeat.shape[0]
    bn = 1000

    def body(x_ref, we, be, wk, bk, wv, bv, wq, bq,
             fv_ref, k_ref, vlo_ref, vhi_ref, q_ref):
        x = x_ref[...]
        fv = _mm(x, we[...]) + be[...]
        fv_ref[...] = fv
        k_ref[...] = _mm(fv, wk[...]) + bk[...]
        v = _mm(fv, wv[...]) + bv[...]
        vlo_ref[...] = v[:, :64]
        vhi_ref[...] = v[:, 64:]
        q_ref[...] = _mm(fv, wq[...]) + bq[...]

    full = lambda a: pl.BlockSpec(a.shape, lambda i: (0,) * a.ndim)
    return pl.pallas_call(
        body,
        grid=(n // bn,),
        in_specs=[pl.BlockSpec((bn, 128), lambda i: (i, 0)),
                  full(w_eff), full(b_eff), full(w_kv), full(b_kv),
                  full(w_vv), full(b_vv), full(w_qv), full(b_qv)],
        out_specs=[pl.BlockSpec((bn, 128), lambda i: (i, 0)),
                   pl.BlockSpec((bn, 64), lambda i: (i, 0)),
                   pl.BlockSpec((bn, 64), lambda i: (i, 0)),
                   pl.BlockSpec((bn, 64), lambda i: (i, 0)),
                   pl.BlockSpec((bn, 64), lambda i: (i, 0))],
        out_shape=[jax.ShapeDtypeStruct((n, 128), jnp.float32),
                   jax.ShapeDtypeStruct((n, 64), jnp.float32),
                   jax.ShapeDtypeStruct((n, 64), jnp.float32),
                   jax.ShapeDtypeStruct((n, 64), jnp.float32),
                   jax.ShapeDtypeStruct((n, 64), jnp.float32)],
    )(vfeat, w_eff, b_eff, w_kv, b_kv, w_vv, b_vv, w_qv, b_qv)


def _prep_e(efeat, w_qe, b_qe):
    m = efeat.shape[0]
    bm = 1000

    def body(x_ref, wq, bq, q_ref):
        q_ref[...] = _mm(x_ref[...], wq[...]) + bq[...]

    full = lambda a: pl.BlockSpec(a.shape, lambda i: (0,) * a.ndim)
    return pl.pallas_call(
        body,
        grid=(m // bm,),
        in_specs=[pl.BlockSpec((bm, 128), lambda i: (i, 0)), full(w_qe), full(b_qe)],
        out_specs=pl.BlockSpec((bm, 64), lambda i: (i, 0)),
        out_shape=jax.ShapeDtypeStruct((m, 64), jnp.float32),
    )(efeat, w_qe, b_qe)


def _attn_scores(gk, gq):
    e = gk.shape[0]
    be = 8000

    def body(k_ref, q_ref, s_ref):
        s = jnp.sum(k_ref[...] * q_ref[...], axis=-1)
        s = jnp.where(s >= 0.0, s, 0.01 * s) * (1.0 / 8.0)
        s_ref[...] = s[None, None, :]

    return pl.pallas_call(
        body,
        grid=(e // be,),
        in_specs=[pl.BlockSpec((be, 64), lambda i: (i, 0)),
                  pl.BlockSpec((be, 64), lambda i: (i, 0))],
        out_specs=pl.BlockSpec((1, 1, be), lambda i: (i, 0, 0)),
        out_shape=jax.ShapeDtypeStruct((e // be, 1, be), jnp.float32),
    )(gk, gq)


def _global_max(s2d):
    nb, _, be = s2d.shape

    def body(s_ref, m_ref):
        i = pl.program_id(0)

        @pl.when(i == 0)
        def _():
            m_ref[...] = jnp.full((1, 1), -jnp.inf, jnp.float32)

        m_ref[...] = jnp.maximum(m_ref[...], jnp.max(s_ref[...]))

    return pl.pallas_call(
        body,
        grid=(nb,),
        in_specs=[pl.BlockSpec((1, 1, be), lambda i: (i, 0, 0))],
        out_specs=pl.BlockSpec((1, 1), lambda i: (0, 0)),
        out_shape=jax.ShapeDtypeStruct((1, 1), jnp.float32),
    )(s2d)


def _exp_u(s2d, gmax):
    nb, _, be = s2d.shape

    def body(s_ref, m_ref, u_ref):
        u_ref[...] = jnp.exp(s_ref[...] - m_ref[0, 0])

    u = pl.pallas_call(
        body,
        grid=(nb,),
        in_specs=[pl.BlockSpec((1, 1, be), lambda i: (i, 0, 0)),
                  pl.BlockSpec((1, 1), lambda i: (0, 0))],
        out_specs=pl.BlockSpec((1, 1, be), lambda i: (i, 0, 0)),
        out_shape=jax.ShapeDtypeStruct((nb, 1, be), jnp.float32),
    )(s2d, gmax)
    return u.reshape(nb * be)


def _post1(num, efeat, ln1_g, ln1_b, w_l1, b_l1, w_l2, b_l2, w_ke, b_ke, w_ve, b_ve):
    m = efeat.shape[0]
    bm = 1000

    def body(o_ref, ef_ref, lg, lb, w1, b1, w2, b2, wk, bk, wv, bv,
             k_ref, vlo_ref, vhi_ref):
        o0 = o_ref[0]
        o1 = o_ref[1]
        d = o0[:, 64:65] + 1e-12
        h = jnp.concatenate([o0[:, :64], o1[:, :64]], axis=1) / d
        x = _ln(h + ef_ref[...], lg[...], lb[...])
        ff = _mm(jnp.maximum(_mm(x, w1[...]) + b1[...], 0.0), w2[...]) + b2[...]
        fe = _ln(ff + x, lg[...], lb[...])
        k_ref[...] = _mm(fe, wk[...]) + bk[...]
        v = _mm(fe, wv[...]) + bv[...]
        vlo_ref[...] = v[:, :64]
        vhi_ref[...] = v[:, 64:]

    full = lambda a: pl.BlockSpec(a.shape, lambda i: (0,) * a.ndim)
    return pl.pallas_call(
        body,
        grid=(m // bm,),
        in_specs=[pl.BlockSpec((2, bm, 80), lambda i: (0, i, 0)),
                  pl.BlockSpec((bm, 128), lambda i: (i, 0)),
                  full(ln1_g), full(ln1_b), full(w_l1), full(b_l1),
                  full(w_l2), full(b_l2), full(w_ke), full(b_ke),
                  full(w_ve), full(b_ve)],
        out_specs=[pl.BlockSpec((bm, 64), lambda i: (i, 0)),
                   pl.BlockSpec((bm, 64), lambda i: (i, 0)),
                   pl.BlockSpec((bm, 64), lambda i: (i, 0))],
        out_shape=[jax.ShapeDtypeStruct((m, 64), jnp.float32),
                   jax.ShapeDtypeStruct((m, 64), jnp.float32),
                   jax.ShapeDtypeStruct((m, 64), jnp.float32)],
    )(num, efeat, ln1_g, ln1_b, w_l1, b_l1, w_l2, b_l2, w_ke, b_ke, w_ve, b_ve)


def _post2(num, feat_v, ln2_g, ln2_b, w_l3, b_l3, w_l4, b_l4,
           w_cls, b_cls, w_mlp, b_mlp):
    n = feat_v.shape[0]
    bn = 1000

    def body(o_ref, fv_ref, lg, lb, w3, b3, w4, b4, wc, bc, wm, bm_,
             pred_ref, fm_ref):
        o0 = o_ref[0]
        o1 = o_ref[1]
        d = o0[:, 64:65] + 1e-12
        h = jnp.concatenate([o0[:, :64], o1[:, :64]], axis=1) / d
        x = _ln(h + fv_ref[...], lg[...], lb[...])
        ff = _mm(jnp.maximum(_mm(x, w3[...]) + b3[...], 0.0), w4[...]) + b4[...]
        fv = _ln(ff + x, lg[...], lb[...])
        pred_ref[...] = _mm(fv, wc[...]) + bc[...]
        fm_ref[...] = _mm(fv, wm[...]) + bm_[...]

    full = lambda a: pl.BlockSpec(a.shape, lambda i: (0,) * a.ndim)
    return pl.pallas_call(
        body,
        grid=(n // bn,),
        in_specs=[pl.BlockSpec((2, bn, 80), lambda i: (0, i, 0)),
                  pl.BlockSpec((bn, 128), lambda i: (i, 0)),
                  full(ln2_g), full(ln2_b), full(w_l3), full(b_l3),
                  full(w_l4), full(b_l4), full(w_cls), full(b_cls),
                  full(w_mlp), full(b_mlp)],
        out_specs=[pl.BlockSpec((bn, 40), lambda i: (i, 0)),
                   pl.BlockSpec((bn, 128), lambda i: (i, 0))],
        out_shape=[jax.ShapeDtypeStruct((n, 40), jnp.float32),
                   jax.ShapeDtypeStruct((n, 128), jnp.float32)],
    )(num, feat_v, ln2_g, ln2_b, w_l3, b_l3, w_l4, b_l4, w_cls, b_cls, w_mlp, b_mlp)


def _sc_gather(table, idx):
    e = idx.shape[0]
    d = table.shape[1]
    per_w = e // NW
    n_grp = per_w // GB

    def body(tab_ref, idx_ref, out_ref, idx_v, rows_v, sem):
        wid = lax.axis_index("s") * NC + lax.axis_index("c")
        base = wid * per_w

        def step(g, carry):
            off = pl.multiple_of(base + g * GB, 8)
            pltpu.sync_copy(idx_ref.at[pl.ds(off, GB)], idx_v)
            descs = []
            for b in range(NBUF):
                descs.append(pltpu.async_copy(
                    tab_ref.at[idx_v.at[pl.ds(b * CH, CH)]],
                    rows_v.at[pl.ds(b * CH, CH)], sem))
            for dd in descs:
                dd.wait()
            pltpu.sync_copy(rows_v, out_ref.at[pl.ds(off, GB)])
            return carry

        lax.fori_loop(0, n_grp, step, 0)

    return pl.kernel(
        body,
        out_type=jax.ShapeDtypeStruct((e, d), jnp.float32),
        mesh=_sc_mesh(),
        compiler_params=pltpu.CompilerParams(use_tc_tiling_on_sc=False),
        scratch_types=[pltpu.VMEM((GB,), jnp.int32),
                       pltpu.VMEM((GB, d), jnp.float32),
                       pltpu.SemaphoreType.DMA],
    )(table, idx)


def _sc_scale_scatter(v_lo, v_hi, u, gidx, sidx2, zeros, n_seg):
    e = u.shape[0]
    nbuf_s = 2
    gb_s = CH * nbuf_s
    per_w = e // NS
    n_grp = per_w // gb_s
    rows_per_tile = n_seg // NS

    def body(vlo_ref, vhi_ref, u_ref, gidx_ref, sidx2_ref, z_ref, out_ref,
             acc, gidx_v, sidx_v, u_v, rows_v, upd_v, sem):
        cid = lax.axis_index("c")
        sid = lax.axis_index("s")
        r0 = sid * rows_per_tile
        pltpu.sync_copy(z_ref.at[pl.ds(r0, rows_per_tile)],
                        acc.at[pl.ds(r0, rows_per_tile)])
        plsc.subcore_barrier()

        def step(g, carry):
            off = pl.multiple_of(sid * per_w + g * gb_s, 8)
            row0 = (sid * per_w) // CH + g * nbuf_s
            pltpu.sync_copy(sidx2_ref.at[pl.ds(row0, nbuf_s)], sidx_v)
            pltpu.sync_copy(gidx_ref.at[pl.ds(off, gb_s)], gidx_v)
            pltpu.sync_copy(u_ref.at[pl.ds(off, gb_s)], u_v)

            @pl.when(cid == 0)
            def _():
                for b in range(nbuf_s):
                    pltpu.async_copy(vlo_ref.at[gidx_v.at[pl.ds(b * CH, CH)]],
                                     rows_v.at[pl.ds(b * CH, CH)], sem).wait()

            @pl.when(cid == 1)
            def _():
                for b in range(nbuf_s):
                    pltpu.async_copy(vhi_ref.at[gidx_v.at[pl.ds(b * CH, CH)]],
                                     rows_v.at[pl.ds(b * CH, CH)], sem).wait()

            def scale(i, carry2):
                row_idx = i * 16 + lax.iota(jnp.int32, 16)
                u16 = u_v[pl.ds(i * 16, 16)]
                for dd in range(64):
                    cidx = jnp.full((16,), dd, jnp.int32)
                    col = plsc.load_gather(rows_v, [row_idx, cidx])
                    plsc.store_scatter(upd_v, [row_idx, cidx], col * u16)
                for dd in range(64, 80):
                    cidx = jnp.full((16,), dd, jnp.int32)
                    plsc.store_scatter(upd_v, [row_idx, cidx], u16)
                return carry2

            lax.fori_loop(0, gb_s // 16, scale, 0)

            for b in range(nbuf_s):
                pltpu.sync_copy(upd_v.at[pl.ds(b * CH, CH)],
                                acc.at[sidx_v.at[b]], add=True)
            return carry

        lax.fori_loop(0, n_grp, step, 0)
        plsc.subcore_barrier()
        pltpu.sync_copy(acc.at[pl.ds(r0, rows_per_tile)],
                        out_ref.at[cid, pl.ds(r0, rows_per_tile)])

    return pl.kernel(
        body,
        out_type=jax.ShapeDtypeStruct((NC, n_seg, 80), jnp.float32),
        mesh=_sc_mesh(),
        compiler_params=pltpu.CompilerParams(use_tc_tiling_on_sc=False,
                                             needs_layout_passes=False),
        scratch_types=[pltpu.VMEM_SHARED((n_seg, 80), jnp.float32),
                       pltpu.VMEM((gb_s,), jnp.int32),
                       pltpu.VMEM((nbuf_s, CH), jnp.int32),
                       pltpu.VMEM((gb_s,), jnp.float32),
                       pltpu.VMEM((gb_s, 64), jnp.float32),
                       pltpu.VMEM((gb_s, 80), jnp.float32),
                       pltpu.SemaphoreType.DMA],
    )(v_lo, v_hi, u, gidx, sidx2, zeros)


def kernel(vfeat, efeat, node_idx, hedge_idx, first_layer, last_layer,
           W_vtx1, b_vtx1, W_vtx, b_vtx, W_kv, b_kv, W_vv, b_vv,
           W_qe, b_qe, W_ke, b_ke, W_ve, b_ve, W_qv, b_qv,
           W_l1, b_l1, W_l2, b_l2, W_l3, b_l3, W_l4, b_l4,
           W_cls, b_cls, W_mlp, b_mlp, ln1_g, ln1_b, ln2_g, ln2_b):
    n = vfeat.shape[0]
    m = efeat.shape[0]
    e = node_idx.shape[0]

    first = jnp.asarray(first_layer) != 0
    w_eff = jnp.where(first, W_vtx1, W_vtx)
    b_eff = jnp.where(first, b_vtx1, b_vtx)

    r2 = lambda b: b.reshape(1, -1)

    feat_v, k_n, v_lo, v_hi, q_v = _prep_v(
        vfeat, w_eff, r2(b_eff), W_kv, r2(b_kv), W_vv, r2(b_vv), W_qv, r2(b_qv))
    q_e = _prep_e(efeat, W_qe, r2(b_qe))

    node_idx2 = node_idx.reshape(e // CH, CH)
    hedge_idx2 = hedge_idx.reshape(e // CH, CH)
    zeros_m = jnp.zeros((m, 80), jnp.float32)
    zeros_n = jnp.zeros((n, 80), jnp.float32)

    gk = _sc_gather(k_n, node_idx)
    gq = _sc_gather(q_e, hedge_idx)
    s2d = _attn_scores(gk, gq)
    gmax = _global_max(s2d)
    u = _exp_u(s2d, gmax)
    num_e = _sc_scale_scatter(v_lo, v_hi, u, node_idx, hedge_idx2, zeros_m, m)
    k_e, ve_lo, ve_hi = _post1(num_e, efeat, r2(ln1_g), r2(ln1_b), W_l1, r2(b_l1),
                               W_l2, r2(b_l2), W_ke, r2(b_ke), W_ve, r2(b_ve))

    gk2 = _sc_gather(k_e, hedge_idx)
    gq2 = _sc_gather(q_v, node_idx)
    s2d2 = _attn_scores(gk2, gq2)
    gmax2 = _global_max(s2d2)
    u2 = _exp_u(s2d2, gmax2)
    num_v = _sc_scale_scatter(ve_lo, ve_hi, u2, hedge_idx, node_idx2, zeros_n, n)
    pred, fm = _post2(num_v, feat_v, r2(ln2_g), r2(ln2_b), W_l3, r2(b_l3),
                      W_l4, r2(b_l4), W_cls, r2(b_cls), W_mlp, r2(b_mlp))

    last = jnp.asarray(last_layer) != 0
    pred = jnp.where(last, pred, 0.0)
    fm = jnp.where(last, fm, 0.0)
    return (pred, fm)


# speedup vs baseline: 1.5021x; 1.5021x over previous
import functools

import jax
import jax.numpy as jnp
import numpy as np
from jax import lax
from jax.experimental import pallas as pl
from jax.experimental.pallas import tpu as pltpu
from jax.experimental.pallas import tpu_sc as plsc

NC = 2
NS = 16
NW = NC * NS
CH = 80
NBUF = 5
GB = CH * NBUF


def _sc_mesh():
    return plsc.VectorSubcoreMesh(
        core_axis_name="c", subcore_axis_name="s", num_cores=NC, num_subcores=NS
    )


def _mm(x, w):
    return lax.dot_general(x, w, (((1,), (1,)), ((), ())),
                           preferred_element_type=jnp.float32)


def _ln(x, g, b):
    mu = jnp.mean(x, axis=-1, keepdims=True)
    var = jnp.mean((x - mu) ** 2, axis=-1, keepdims=True)
    return (x - mu) * lax.rsqrt(var + 1e-5) * g + b


def _prep_v(vfeat, w_eff, b_eff, w_kv, b_kv, w_vv, b_vv, w_qv, b_qv):
    n = vfeat.shape[0]
    bn = 1000

    def body(x_ref, we, be, wk, bk, wv, bv, wq, bq, fv_ref, k_ref, v_ref, q_ref):
        x = x_ref[...]
        fv = _mm(x, we[...]) + be[...]
        fv_ref[...] = fv
        k_ref[...] = _mm(fv, wk[...]) + bk[...]
        v_ref[...] = _mm(fv, wv[...]) + bv[...]
        q_ref[...] = _mm(fv, wq[...]) + bq[...]

    full = lambda a: pl.BlockSpec(a.shape, lambda i: (0,) * a.ndim)
    return pl.pallas_call(
        body,
        grid=(n // bn,),
        in_specs=[pl.BlockSpec((bn, 128), lambda i: (i, 0)),
                  full(w_eff), full(b_eff), full(w_kv), full(b_kv),
                  full(w_vv), full(b_vv), full(w_qv), full(b_qv)],
        out_specs=[pl.BlockSpec((bn, 128), lambda i: (i, 0)),
                   pl.BlockSpec((bn, 64), lambda i: (i, 0)),
                   pl.BlockSpec((bn, 128), lambda i: (i, 0)),
                   pl.BlockSpec((bn, 64), lambda i: (i, 0))],
        out_shape=[jax.ShapeDtypeStruct((n, 128), jnp.float32),
                   jax.ShapeDtypeStruct((n, 64), jnp.float32),
                   jax.ShapeDtypeStruct((n, 128), jnp.float32),
                   jax.ShapeDtypeStruct((n, 64), jnp.float32)],
    )(vfeat, w_eff, b_eff, w_kv, b_kv, w_vv, b_vv, w_qv, b_qv)


def _prep_e(efeat, w_qe, b_qe):
    m = efeat.shape[0]
    bm = 1000

    def body(x_ref, wq, bq, q_ref):
        q_ref[...] = _mm(x_ref[...], wq[...]) + bq[...]

    full = lambda a: pl.BlockSpec(a.shape, lambda i: (0,) * a.ndim)
    return pl.pallas_call(
        body,
        grid=(m // bm,),
        in_specs=[pl.BlockSpec((bm, 128), lambda i: (i, 0)), full(w_qe), full(b_qe)],
        out_specs=pl.BlockSpec((bm, 64), lambda i: (i, 0)),
        out_shape=jax.ShapeDtypeStruct((m, 64), jnp.float32),
    )(efeat, w_qe, b_qe)


def _attn_scores(gk, gq):
    e = gk.shape[0]
    be = 8000

    def body(k_ref, q_ref, s_ref):
        s = jnp.sum(k_ref[...] * q_ref[...], axis=-1)
        s = jnp.where(s >= 0.0, s, 0.01 * s) * (1.0 / 8.0)
        s_ref[...] = s[None, None, :]

    return pl.pallas_call(
        body,
        grid=(e // be,),
        in_specs=[pl.BlockSpec((be, 64), lambda i: (i, 0)),
                  pl.BlockSpec((be, 64), lambda i: (i, 0))],
        out_specs=pl.BlockSpec((1, 1, be), lambda i: (i, 0, 0)),
        out_shape=jax.ShapeDtypeStruct((e // be, 1, be), jnp.float32),
    )(gk, gq)


def _global_max(s2d):
    nb, _, be = s2d.shape

    def body(s_ref, m_ref):
        i = pl.program_id(0)

        @pl.when(i == 0)
        def _():
            m_ref[...] = jnp.full((1, 1), -jnp.inf, jnp.float32)

        m_ref[...] = jnp.maximum(m_ref[...], jnp.max(s_ref[...]))

    return pl.pallas_call(
        body,
        grid=(nb,),
        in_specs=[pl.BlockSpec((1, 1, be), lambda i: (i, 0, 0))],
        out_specs=pl.BlockSpec((1, 1), lambda i: (0, 0)),
        out_shape=jax.ShapeDtypeStruct((1, 1), jnp.float32),
    )(s2d)


def _make_updates(s2d, gmax, gv):
    e = gv.shape[0]
    nb, _, bw = s2d.shape
    be = bw

    def body(s_ref, m_ref, v_ref, u0_ref, u1_ref):
        u = jnp.exp(s_ref[0, 0, :] - m_ref[0, 0])
        uc = u[:, None]
        v = v_ref[...]
        ub = jnp.broadcast_to(uc, (be, 16))
        u0_ref[...] = jnp.concatenate([uc * v[:, :64], ub], axis=1)
        u1_ref[...] = jnp.concatenate([uc * v[:, 64:], ub], axis=1)

    return pl.pallas_call(
        body,
        grid=(e // be,),
        in_specs=[pl.BlockSpec((1, 1, be), lambda i: (i, 0, 0)),
                  pl.BlockSpec((1, 1), lambda i: (0, 0)),
                  pl.BlockSpec((be, 128), lambda i: (i, 0))],
        out_specs=[pl.BlockSpec((be, 80), lambda i: (i, 0)),
                   pl.BlockSpec((be, 80), lambda i: (i, 0))],
        out_shape=[jax.ShapeDtypeStruct((e, 80), jnp.float32),
                   jax.ShapeDtypeStruct((e, 80), jnp.float32)],
    )(s2d, gmax, gv)


def _post1(num, efeat, ln1_g, ln1_b, w_l1, b_l1, w_l2, b_l2, w_ke, b_ke, w_ve, b_ve):
    m = efeat.shape[0]
    bm = 1000

    def body(o_ref, ef_ref, lg, lb, w1, b1, w2, b2, wk, bk, wv, bv, k_ref, v_ref):
        o0 = o_ref[0]
        o1 = o_ref[1]
        d = o0[:, 64:65] + 1e-12
        h = jnp.concatenate([o0[:, :64], o1[:, :64]], axis=1) / d
        x = _ln(h + ef_ref[...], lg[...], lb[...])
        ff = _mm(jnp.maximum(_mm(x, w1[...]) + b1[...], 0.0), w2[...]) + b2[...]
        fe = _ln(ff + x, lg[...], lb[...])
        k_ref[...] = _mm(fe, wk[...]) + bk[...]
        v_ref[...] = _mm(fe, wv[...]) + bv[...]

    full = lambda a: pl.BlockSpec(a.shape, lambda i: (0,) * a.ndim)
    return pl.pallas_call(
        body,
        grid=(m // bm,),
        in_specs=[pl.BlockSpec((2, bm, 80), lambda i: (0, i, 0)),
                  pl.BlockSpec((bm, 128), lambda i: (i, 0)),
                  full(ln1_g), full(ln1_b), full(w_l1), full(b_l1),
                  full(w_l2), full(b_l2), full(w_ke), full(b_ke),
                  full(w_ve), full(b_ve)],
        out_specs=[pl.BlockSpec((bm, 64), lambda i: (i, 0)),
                   pl.BlockSpec((bm, 128), lambda i: (i, 0))],
        out_shape=[jax.ShapeDtypeStruct((m, 64), jnp.float32),
                   jax.ShapeDtypeStruct((m, 128), jnp.float32)],
    )(num, efeat, ln1_g, ln1_b, w_l1, b_l1, w_l2, b_l2, w_ke, b_ke, w_ve, b_ve)


def _post2(num, feat_v, ln2_g, ln2_b, w_l3, b_l3, w_l4, b_l4,
           w_cls, b_cls, w_mlp, b_mlp):
    n = feat_v.shape[0]
    bn = 1000

    def body(o_ref, fv_ref, lg, lb, w3, b3, w4, b4, wc, bc, wm, bm_,
             pred_ref, fm_ref):
        o0 = o_ref[0]
        o1 = o_ref[1]
        d = o0[:, 64:65] + 1e-12
        h = jnp.concatenate([o0[:, :64], o1[:, :64]], axis=1) / d
        x = _ln(h + fv_ref[...], lg[...], lb[...])
        ff = _mm(jnp.maximum(_mm(x, w3[...]) + b3[...], 0.0), w4[...]) + b4[...]
        fv = _ln(ff + x, lg[...], lb[...])
        pred_ref[...] = _mm(fv, wc[...]) + bc[...]
        fm_ref[...] = _mm(fv, wm[...]) + bm_[...]

    full = lambda a: pl.BlockSpec(a.shape, lambda i: (0,) * a.ndim)
    return pl.pallas_call(
        body,
        grid=(n // bn,),
        in_specs=[pl.BlockSpec((2, bn, 80), lambda i: (0, i, 0)),
                  pl.BlockSpec((bn, 128), lambda i: (i, 0)),
                  full(ln2_g), full(ln2_b), full(w_l3), full(b_l3),
                  full(w_l4), full(b_l4), full(w_cls), full(b_cls),
                  full(w_mlp), full(b_mlp)],
        out_specs=[pl.BlockSpec((bn, 40), lambda i: (i, 0)),
                   pl.BlockSpec((bn, 128), lambda i: (i, 0))],
        out_shape=[jax.ShapeDtypeStruct((n, 40), jnp.float32),
                   jax.ShapeDtypeStruct((n, 128), jnp.float32)],
    )(num, feat_v, ln2_g, ln2_b, w_l3, b_l3, w_l4, b_l4, w_cls, b_cls, w_mlp, b_mlp)


def _sc_gather(table, idx):
    e = idx.shape[0]
    d = table.shape[1]
    per_w = e // NW
    n_grp = per_w // GB

    def body(tab_ref, idx_ref, out_ref, idx_v, rows_v, sem):
        wid = lax.axis_index("s") * NC + lax.axis_index("c")
        base = wid * per_w

        def step(g, carry):
            off = pl.multiple_of(base + g * GB, 8)
            pltpu.sync_copy(idx_ref.at[pl.ds(off, GB)], idx_v)
            descs = []
            for b in range(NBUF):
                descs.append(pltpu.async_copy(
                    tab_ref.at[idx_v.at[pl.ds(b * CH, CH)]],
                    rows_v.at[pl.ds(b * CH, CH)], sem))
            for dd in descs:
                dd.wait()
            pltpu.sync_copy(rows_v, out_ref.at[pl.ds(off, GB)])
            return carry

        lax.fori_loop(0, n_grp, step, 0)

    return pl.kernel(
        body,
        out_type=jax.ShapeDtypeStruct((e, d), jnp.float32),
        mesh=_sc_mesh(),
        compiler_params=pltpu.CompilerParams(use_tc_tiling_on_sc=False),
        scratch_types=[pltpu.VMEM((GB,), jnp.int32),
                       pltpu.VMEM((GB, d), jnp.float32),
                       pltpu.SemaphoreType.DMA],
    )(table, idx)


def _sc_scatter_add(upd0, upd1, idx2, zeros, n_seg):
    e = upd0.shape[0]
    per_w = e // NS
    n_grp = per_w // CH
    rows_per_tile = n_seg // NS

    def body(u0_ref, u1_ref, idx2_ref, z_ref, out_ref,
             acc, idxs_v, upd_v, sem0, sem1):
        cid = lax.axis_index("c")
        sid = lax.axis_index("s")
        sems = (sem0, sem1)
        r0 = sid * rows_per_tile
        pltpu.sync_copy(z_ref.at[pl.ds(r0, rows_per_tile)],
                        acc.at[pl.ds(r0, rows_per_tile)])
        plsc.subcore_barrier()
        row_base = (sid * per_w) // CH

        def stage(p, g):
            off = pl.multiple_of(sid * per_w + g * CH, 8)
            pltpu.async_copy(idx2_ref.at[pl.ds(row_base + g, 1)],
                             idxs_v.at[pl.ds(p, 1)], sems[p])

            @pl.when(cid == 0)
            def _():
                pltpu.async_copy(u0_ref.at[pl.ds(off, CH)],
                                 upd_v.at[p], sems[p])

            @pl.when(cid == 1)
            def _():
                pltpu.async_copy(u1_ref.at[pl.ds(off, CH)],
                                 upd_v.at[p], sems[p])

        def wait_stage(p):
            pltpu.make_async_copy(idx2_ref.at[pl.ds(0, 1)],
                                  idxs_v.at[pl.ds(p, 1)], sems[p]).wait()
            pltpu.make_async_copy(u0_ref.at[pl.ds(0, CH)],
                                  upd_v.at[p], sems[p]).wait()

        stage(0, 0)

        def step(g2, carry):
            for p in range(2):
                g = g2 * 2 + p
                wait_stage(p)

                @pl.when(g + 1 < n_grp)
                def _():
                    stage(1 - p, g + 1)

                pltpu.sync_copy(upd_v.at[p], acc.at[idxs_v.at[p]], add=True)
            return carry

        lax.fori_loop(0, n_grp // 2, step, 0)
        plsc.subcore_barrier()
        pltpu.sync_copy(acc.at[pl.ds(r0, rows_per_tile)],
                        out_ref.at[cid, pl.ds(r0, rows_per_tile)])

    return pl.kernel(
        body,
        out_type=jax.ShapeDtypeStruct((NC, n_seg, 80), jnp.float32),
        mesh=_sc_mesh(),
        compiler_params=pltpu.CompilerParams(use_tc_tiling_on_sc=False),
        scratch_types=[pltpu.VMEM_SHARED((n_seg, 80), jnp.float32),
                       pltpu.VMEM((2, CH), jnp.int32),
                       pltpu.VMEM((2, CH, 80), jnp.float32),
                       pltpu.SemaphoreType.DMA,
                       pltpu.SemaphoreType.DMA],
    )(upd0, upd1, idx2, zeros)


def kernel(vfeat, efeat, node_idx, hedge_idx, first_layer, last_layer,
           W_vtx1, b_vtx1, W_vtx, b_vtx, W_kv, b_kv, W_vv, b_vv,
           W_qe, b_qe, W_ke, b_ke, W_ve, b_ve, W_qv, b_qv,
           W_l1, b_l1, W_l2, b_l2, W_l3, b_l3, W_l4, b_l4,
           W_cls, b_cls, W_mlp, b_mlp, ln1_g, ln1_b, ln2_g, ln2_b):
    n = vfeat.shape[0]
    m = efeat.shape[0]
    e = node_idx.shape[0]

    first = jnp.asarray(first_layer) != 0
    w_eff = jnp.where(first, W_vtx1, W_vtx)
    b_eff = jnp.where(first, b_vtx1, b_vtx)

    r2 = lambda b: b.reshape(1, -1)

    feat_v, k_n, v_n, q_v = _prep_v(
        vfeat, w_eff, r2(b_eff), W_kv, r2(b_kv), W_vv, r2(b_vv), W_qv, r2(b_qv))
    q_e = _prep_e(efeat, W_qe, r2(b_qe))

    node_idx2 = node_idx.reshape(e // CH, CH)
    hedge_idx2 = hedge_idx.reshape(e // CH, CH)
    zeros_m = jnp.zeros((m, 80), jnp.float32)
    zeros_n = jnp.zeros((n, 80), jnp.float32)

    gk = _sc_gather(k_n, node_idx)
    gq = _sc_gather(q_e, hedge_idx)
    s2d = _attn_scores(gk, gq)
    gmax = _global_max(s2d)
    gv = _sc_gather(v_n, node_idx)
    u0, u1 = _make_updates(s2d, gmax, gv)
    num_e = _sc_scatter_add(u0, u1, hedge_idx2, zeros_m, m)
    k_e, v_e = _post1(num_e, efeat, r2(ln1_g), r2(ln1_b), W_l1, r2(b_l1),
                      W_l2, r2(b_l2), W_ke, r2(b_ke), W_ve, r2(b_ve))

    gk2 = _sc_gather(k_e, hedge_idx)
    gq2 = _sc_gather(q_v, node_idx)
    s2d2 = _attn_scores(gk2, gq2)
    gmax2 = _global_max(s2d2)
    gv2 = _sc_gather(v_e, hedge_idx)
    u20, u21 = _make_updates(s2d2, gmax2, gv2)
    num_v = _sc_scatter_add(u20, u21, node_idx2, zeros_n, n)
    pred, fm = _post2(num_v, feat_v, r2(ln2_g), r2(ln2_b), W_l3, r2(b_l3),
                      W_l4, r2(b_l4), W_cls, r2(b_cls), W_mlp, r2(b_mlp))

    last = jnp.asarray(last_layer) != 0
    pred = jnp.where(last, pred, 0.0)
    fm = jnp.where(last, fm, 0.0)
    return (pred, fm)
